# baseline (speedup 1.0000x reference)
import functools

import jax
import jax.numpy as jnp
from jax.experimental import pallas as pl
from jax.experimental.pallas import tpu as pltpu


def _copy_kernel(x_ref, w1_ref, b1_ref, w2_ref, b2_ref, o_ref):
    o_ref[...] = x_ref[...]


def kernel(x_nchw, w1, b1, w2, b2):
    n, c, h, w = x_nchw.shape
    hw = h * w
    mid = w1.shape[0]
    x3 = x_nchw.reshape(n, c, hw)
    b = 4

    out = pl.pallas_call(
        _copy_kernel,
        grid=(n // b,),
        in_specs=[
            pl.BlockSpec((b, c, hw), lambda i: (i, 0, 0)),
            pl.BlockSpec((mid, c), lambda i: (0, 0)),
            pl.BlockSpec((1, mid), lambda i: (0, 0)),
            pl.BlockSpec((c, mid), lambda i: (0, 0)),
            pl.BlockSpec((1, c), lambda i: (0, 0)),
        ],
        out_specs=pl.BlockSpec((b, c, hw), lambda i: (i, 0, 0)),
        out_shape=jax.ShapeDtypeStruct((n, c, hw), x_nchw.dtype),
        compiler_params=pltpu.CompilerParams(
            dimension_semantics=("arbitrary",),
            vmem_limit_bytes=60 * 1024 * 1024),
    )(x3, w1, b1.reshape(1, mid), w2, b2.reshape(1, c))
    return out.reshape(n, c, h, w)
